# baseline (speedup 1.0000x reference)
import functools

import jax
import jax.numpy as jnp
from jax import lax
from jax.experimental import pallas as pl
from jax.experimental.pallas import tpu as pltpu
from jax.experimental.pallas import tpu_sc as plsc

B = 16384
F = 32
ROWS_PER_LINE = 4
NW = 32
BPW = B // NW
CHUNK = 128
NCHUNK = BPW // CHUNK
NGROUP = CHUNK // 16


def _gmf_body(user_hbm, item_hbm, eu_hbm, ei_hbm, w_hbm, bias_hbm, out_hbm,
              uidx, iidx, uline, iline, ubuf, ibuf, wv, bv, outv,
              sems, sem_out):
    nc = 2
    wid = lax.axis_index("s") * nc + lax.axis_index("c")

    pltpu.sync_copy(user_hbm.at[pl.ds(wid * NCHUNK, NCHUNK)], uidx)
    pltpu.sync_copy(item_hbm.at[pl.ds(wid * NCHUNK, NCHUNK)], iidx)
    pltpu.sync_copy(w_hbm, wv)
    pltpu.sync_copy(bias_hbm, bv)

    for j in range(NCHUNK):
        for v in range(CHUNK // 16):
            s = pl.ds(v * 16, 16)
            uline[j, s] = lax.shift_right_logical(uidx[j, s], 2)
            iline[j, s] = lax.shift_right_logical(iidx[j, s], 2)

    def fire(j, slot):
        cu = pltpu.async_copy(eu_hbm.at[uline.at[j]], ubuf.at[slot], sems.at[slot, 0])
        ci = pltpu.async_copy(ei_hbm.at[iline.at[j]], ibuf.at[slot], sems.at[slot, 1])
        return cu, ci

    w_lo = wv[pl.ds(0, 16)]
    w_hi = wv[pl.ds(16, 16)]
    wcols = [w_lo[f] if f < 16 else w_hi[f - 16] for f in range(F)]
    bias = bv[...]

    pending = fire(0, 0)
    for j in range(NCHUNK):
        nxt = None
        if j + 1 < NCHUNK:
            nxt = fire(j + 1, (j + 1) % 2)
        pending[0].wait()
        pending[1].wait()
        slot = j % 2

        def group(g, carry):
            row_idx = g * 16 + lax.iota(jnp.int32, 16)
            s = pl.ds(g * 16, 16)
            off_u = (uidx[j, s] & 3) * 32
            off_i = (iidx[j, s] & 3) * 32
            acc = bias
            for f in range(F):
                u = plsc.load_gather(ubuf.at[slot], [row_idx, off_u + f])
                iv = plsc.load_gather(ibuf.at[slot], [row_idx, off_i + f])
                acc = acc + u * iv * wcols[f]
            outv[pl.ds(j * CHUNK + g * 16, 16)] = acc
            return carry

        lax.fori_loop(0, NGROUP, group, 0)
        pending = nxt

    pltpu.sync_copy(outv, out_hbm.at[pl.ds(wid * BPW, BPW)])


@functools.partial(jax.jit, static_argnames=())
def _gmf(user2, item2, eu_lines, ei_lines, w_vec, bias_vec):
    mesh = plsc.VectorSubcoreMesh(core_axis_name="c", subcore_axis_name="s",
                                  num_cores=2, num_subcores=16)
    kern = pl.kernel(
        _gmf_body,
        out_type=jax.ShapeDtypeStruct((B,), jnp.float32),
        mesh=mesh,
        compiler_params=pltpu.CompilerParams(needs_layout_passes=False),
        scratch_types=[
            pltpu.VMEM((NCHUNK, CHUNK), jnp.int32),
            pltpu.VMEM((NCHUNK, CHUNK), jnp.int32),
            pltpu.VMEM((NCHUNK, CHUNK), jnp.int32),
            pltpu.VMEM((NCHUNK, CHUNK), jnp.int32),
            pltpu.VMEM((2, CHUNK, 128), jnp.float32),
            pltpu.VMEM((2, CHUNK, 128), jnp.float32),
            pltpu.VMEM((F,), jnp.float32),
            pltpu.VMEM((16,), jnp.float32),
            pltpu.VMEM((BPW,), jnp.float32),
            pltpu.SemaphoreType.DMA((2, 2)),
            pltpu.SemaphoreType.DMA,
        ],
    )
    return kern(user2, item2, eu_lines, ei_lines, w_vec, bias_vec)


def kernel(user, item, embed_user, embed_item, W, b):
    user2 = user.reshape(NW * NCHUNK, CHUNK)
    item2 = item.reshape(NW * NCHUNK, CHUNK)
    eu_lines = embed_user.reshape(-1, ROWS_PER_LINE * F)
    ei_lines = embed_item.reshape(-1, ROWS_PER_LINE * F)
    w_vec = W.reshape(F)
    bias_vec = jnp.broadcast_to(b, (16,))
    return _gmf(user2, item2, eu_lines, ei_lines, w_vec, bias_vec)


# speedup vs baseline: 3.2827x; 3.2827x over previous
import functools

import jax
import jax.numpy as jnp
from jax import lax
from jax.experimental import pallas as pl
from jax.experimental.pallas import tpu as pltpu
from jax.experimental.pallas import tpu_sc as plsc

B = 16384
F = 32
NW = 32
NROW = 1000000
NTC = 7813
TAIL = 999936
CTC = 4
NCH = 62
CAP = 768
BCAP = 128
CHW = CTC * 128
OCAP = 64


def _popc(mask):
    return plsc.all_reduce_population_count(mask)[0]


def _gather_body(user_hbm, item_hbm, euT, eiT, tailu_hbm, taili_hbm, gu, gi,
                 idxstage, fl_r, fl_b, l1_r, l1_b, l2_r, l2_b, bk_r, bk_b,
                 cur_r, cur_b, tailbuf, outstage, cbuf, sems, osem):
    nc = 2
    wid = lax.axis_index("s") * nc + lax.axis_index("c")

    lo_tc = (wid * NTC) >> 5
    hi_tc = ((wid + 1) * NTC) >> 5
    lo_r = lo_tc * 128
    hi_r = jnp.minimum(hi_tc * 128, NROW)

    iota = lax.iota(jnp.int32, 16)
    neg1 = jnp.full((16,), -1, jnp.int32)

    max_tc = (NROW - CHW) // 128

    def fire(c):
        base = pl.multiple_of(
            jnp.minimum(lo_tc + CTC * c, max_tc) * 128, 128)
        for tab, src in ((0, euT), (1, eiT)):
            for fb in range(4):
                for t in range(CTC):
                    pltpu.async_copy(
                        src.at[pl.ds(fb * 8, 8), pl.ds(base + t * 128, 128)],
                        cbuf.at[tab, fb, t], sems.at[0])

    def wait_chunk():
        for tab in range(2):
            for fb in range(4):
                for t in range(CTC):
                    pltpu.make_async_copy(
                        euT.at[pl.ds(0, 8), pl.ds(0, 128)],
                        cbuf.at[tab, fb, t], sems.at[0]).wait()

    pltpu.sync_copy(tailu_hbm, tailbuf.at[0])
    pltpu.sync_copy(taili_hbm, tailbuf.at[1])

    def prefill(k, carry):
        bk_r[pl.ds(k * 16, 16)] = neg1
        return carry

    lax.fori_loop(0, (16 * BCAP) // 16, prefill, 0)

    cnts = []
    for tab, src_hbm in ((0, user_hbm), (1, item_hbm)):
        pltpu.sync_copy(src_hbm, idxstage)

        def bin_step(t, cnt, tab=tab):
            off = t * 16
            bvec = off + iota
            v = idxstage[pl.ds(off, 16)]
            m = (v >= lo_r) & (v < hi_r)
            plsc.store_compressed(fl_r.at[pl.ds(tab * CAP + cnt, 16)], v,
                                  mask=m)
            plsc.store_compressed(fl_b.at[pl.ds(tab * CAP + cnt, 16)], bvec,
                                  mask=m)
            return cnt + _popc(m)

        cnts.append(lax.fori_loop(0, B // 16, bin_step, jnp.int32(0)))

    def split(src_r, src_b, s_base, s_cnt, dst_r, dst_b, d0, d1, mid, nv):
        def body(k, carry):
            cl, cr = carry
            pos = k * 16
            rv = src_r[pl.ds(s_base + pos, 16)]
            bv = src_b[pl.ds(s_base + pos, 16)]
            valid = (pos + iota) < s_cnt
            ml = valid & (rv < mid)
            mr = valid & (rv >= mid)
            plsc.store_compressed(dst_r.at[pl.ds(d0 + cl, 16)], rv, mask=ml)
            plsc.store_compressed(dst_b.at[pl.ds(d0 + cl, 16)], bv, mask=ml)
            plsc.store_compressed(dst_r.at[pl.ds(d1 + cr, 16)], rv, mask=mr)
            plsc.store_compressed(dst_b.at[pl.ds(d1 + cr, 16)], bv, mask=mr)
            return cl + _popc(ml), cr + _popc(mr)

        return lax.fori_loop(0, nv, body, (jnp.int32(0), jnp.int32(0)))

    for tab in range(2):
        c0, c1 = split(fl_r, fl_b, tab * CAP, cnts[tab],
                       l1_r, l1_b, tab * 768, tab * 768 + 384,
                       lo_r + 16384, CAP // 16)
        l2c = []
        for h, ch in ((0, c0), (1, c1)):
            mid = lo_r + h * 16384 + 8192
            a, b_ = split(l1_r, l1_b, tab * 768 + h * 384, ch,
                          l2_r, l2_b, tab * 768 + h * 384,
                          tab * 768 + h * 384 + 192, mid, 384 // 16)
            l2c += [a, b_]
        for q in range(4):
            mid = lo_r + q * 8192 + 4096
            split(l2_r, l2_b, tab * 768 + q * 192, l2c[q],
                  bk_r, bk_b, (tab * 8 + 2 * q) * BCAP,
                  (tab * 8 + 2 * q + 1) * BCAP, mid, 192 // 16)

    fbv_lo = iota >> 3
    fbv_hi = 2 + (iota >> 3)
    subv = iota & 7

    def extract_entries(tab, base, ccnt, from_tail):
        out_ref = gu if tab == 0 else gi

        def entry_vreg(e, carry):
            rv = cur_r[pl.ds(tab * BCAP + e * 16, 16)]
            bv = cur_b[pl.ds(tab * BCAP + e * 16, 16)]
            for k in range(16):
                @pl.when(e * 16 + k < ccnt)
                def _():
                    r = rv[k]
                    bpos = bv[k]
                    n = (tab * OCAP + e * 16 + k) * F
                    if from_tail:
                        rloc = jnp.full((16,), r - TAIL, jnp.int32)
                        lo = plsc.load_gather(tailbuf.at[tab], [iota, rloc])
                        hi = plsc.load_gather(tailbuf.at[tab],
                                              [16 + iota, rloc])
                    else:
                        rl = r - base
                        tcv = jnp.full((16,), rl >> 7, jnp.int32)
                        lnv = jnp.full((16,), rl & 127, jnp.int32)
                        lo = plsc.load_gather(cbuf.at[tab],
                                              [fbv_lo, tcv, subv, lnv])
                        hi = plsc.load_gather(cbuf.at[tab],
                                              [fbv_hi, tcv, subv, lnv])
                    outstage[pl.ds(n, 16)] = lo
                    outstage[pl.ds(n + 16, 16)] = hi
                    pltpu.async_copy(
                        outstage.at[pl.ds(n, F)],
                        out_ref.at[pl.ds(pl.multiple_of(bpos * F, F), F)],
                        osem)
            return carry

        lax.fori_loop(0, (ccnt + 15) >> 4, entry_vreg, 0)

    def drain_out(n):
        def body(_, carry):
            pltpu.make_async_copy(outstage.at[pl.ds(0, F)],
                                  gu.at[pl.ds(0, F)], osem).wait()
            return carry
        lax.fori_loop(0, n, body, 0)

    def process(c):
        base = jnp.minimum(lo_tc + CTC * c, max_tc) * 128
        bkt = c >> 3
        fired = jnp.int32(0)
        for tab in range(2):
            boff = (tab * 8) * BCAP
            ccnt = jnp.int32(0)
            for k in range(BCAP // 16):
                pos = k * 16
                rv = bk_r[pl.ds(boff + bkt * BCAP + pos, 16)]
                bv = bk_b[pl.ds(boff + bkt * BCAP + pos, 16)]
                m = (rv >= base) & (rv < base + CHW)
                plsc.store_compressed(
                    cur_r.at[pl.ds(tab * BCAP + ccnt, 16)], rv, mask=m)
                plsc.store_compressed(
                    cur_b.at[pl.ds(tab * BCAP + ccnt, 16)], bv, mask=m)
                ccnt = ccnt + _popc(m)
            ccnt = jnp.minimum(ccnt, OCAP)
            extract_entries(tab, base, ccnt, False)
            fired = fired + ccnt
        drain_out(fired)

    def step(c, carry):
        fire(c)
        wait_chunk()
        process(c)
        return carry

    lax.fori_loop(0, NCH, step, 0)

    for tab in range(2):
        boff = (tab * 8 + 7) * BCAP
        ccnt = jnp.int32(0)
        for k in range(BCAP // 16):
            pos = k * 16
            rv = bk_r[pl.ds(boff + pos, 16)]
            bv = bk_b[pl.ds(boff + pos, 16)]
            m = rv >= TAIL
            plsc.store_compressed(cur_r.at[pl.ds(tab * BCAP + ccnt, 16)], rv,
                                  mask=m)
            plsc.store_compressed(cur_b.at[pl.ds(tab * BCAP + ccnt, 16)], bv,
                                  mask=m)
            ccnt = ccnt + _popc(m)
        ccnt = jnp.minimum(ccnt, OCAP)
        extract_entries(tab, 0, ccnt, True)
        drain_out(ccnt)


def _make_gather_kernel():
    mesh = plsc.VectorSubcoreMesh(core_axis_name="c", subcore_axis_name="s",
                                  num_cores=2, num_subcores=16)
    return pl.kernel(
        _gather_body,
        out_type=(jax.ShapeDtypeStruct((B * F,), jnp.float32),
                  jax.ShapeDtypeStruct((B * F,), jnp.float32)),
        mesh=mesh,
        compiler_params=pltpu.CompilerParams(needs_layout_passes=False),
        scratch_types=[
            pltpu.VMEM((B,), jnp.int32),
            pltpu.VMEM((2 * CAP,), jnp.int32),
            pltpu.VMEM((2 * CAP,), jnp.int32),
            pltpu.VMEM((2 * 768,), jnp.int32),
            pltpu.VMEM((2 * 768,), jnp.int32),
            pltpu.VMEM((2 * 768,), jnp.int32),
            pltpu.VMEM((2 * 768,), jnp.int32),
            pltpu.VMEM((16 * BCAP,), jnp.int32),
            pltpu.VMEM((16 * BCAP,), jnp.int32),
            pltpu.VMEM((2 * BCAP,), jnp.int32),
            pltpu.VMEM((2 * BCAP,), jnp.int32),
            pltpu.VMEM((2, F, 128), jnp.float32),
            pltpu.VMEM((2 * OCAP * F,), jnp.float32),
            pltpu.VMEM((2, 4, CTC, 8, 128), jnp.float32),
            pltpu.SemaphoreType.DMA((1,)),
            pltpu.SemaphoreType.DMA,
        ],
    )


def _compute_body(gu, gi, w_hbm, bias_hbm, out_hbm,
                  ubuf, ibuf, wv, bv, outv, sem_unused):
    nc = 2
    wid = lax.axis_index("s") * nc + lax.axis_index("c")
    bpw = B // NW

    pltpu.sync_copy(gu.at[pl.ds(wid * bpw * F, bpw * F)], ubuf)
    pltpu.sync_copy(gi.at[pl.ds(wid * bpw * F, bpw * F)], ibuf)
    pltpu.sync_copy(w_hbm, wv)
    pltpu.sync_copy(bias_hbm, bv)

    w_lo = wv[pl.ds(0, 16)]
    w_hi = wv[pl.ds(16, 16)]
    wcols = [w_lo[f] if f < 16 else w_hi[f - 16] for f in range(F)]
    bias = bv[...]
    iota32 = lax.iota(jnp.int32, 16) * F

    def group(g, carry):
        idx0 = g * (16 * F) + iota32
        acc = bias
        for f in range(F):
            u = plsc.load_gather(ubuf, [idx0 + f])
            iv = plsc.load_gather(ibuf, [idx0 + f])
            acc = acc + u * iv * wcols[f]
        outv[pl.ds(g * 16, 16)] = acc
        return carry

    lax.fori_loop(0, bpw // 16, group, 0)
    pltpu.sync_copy(outv, out_hbm.at[pl.ds(wid * bpw, bpw)])


def _make_compute_kernel():
    mesh = plsc.VectorSubcoreMesh(core_axis_name="c", subcore_axis_name="s",
                                  num_cores=2, num_subcores=16)
    bpw = B // NW
    return pl.kernel(
        _compute_body,
        out_type=jax.ShapeDtypeStruct((B,), jnp.float32),
        mesh=mesh,
        compiler_params=pltpu.CompilerParams(needs_layout_passes=False),
        scratch_types=[
            pltpu.VMEM((bpw * F,), jnp.float32),
            pltpu.VMEM((bpw * F,), jnp.float32),
            pltpu.VMEM((F,), jnp.float32),
            pltpu.VMEM((16,), jnp.float32),
            pltpu.VMEM((bpw,), jnp.float32),
            pltpu.SemaphoreType.DMA,
        ],
    )


@functools.partial(jax.jit, static_argnames=())
def _gmf(user, item, euT, eiT, tail_u, tail_i, w_vec, bias_vec):
    gu, gi = _make_gather_kernel()(user, item, euT, eiT, tail_u, tail_i)
    return _make_compute_kernel()(gu, gi, w_vec, bias_vec)


def kernel(user, item, embed_user, embed_item, W, b):
    euT = embed_user.T
    eiT = embed_item.T
    tail_u = jnp.pad(euT[:, TAIL:], ((0, 0), (0, 64)))
    tail_i = jnp.pad(eiT[:, TAIL:], ((0, 0), (0, 64)))
    w_vec = W.reshape(F)
    bias_vec = jnp.broadcast_to(b, (16,))
    return _gmf(user, item, euT, eiT, tail_u, tail_i, w_vec, bias_vec)


# speedup vs baseline: 3.3173x; 1.0105x over previous
import functools

import jax
import jax.numpy as jnp
from jax import lax
from jax.experimental import pallas as pl
from jax.experimental.pallas import tpu as pltpu
from jax.experimental.pallas import tpu_sc as plsc

B = 16384
F = 32
NW = 32
NROW = 1000000
NTC = 7813
TAIL = 999936
CTC = 4
NCH = 62
CAP = 768
BCAP = 128
CHW = CTC * 128
OCAP = 64


def _popc(mask):
    return plsc.all_reduce_population_count(mask)[0]


def _gather_body(user_hbm, item_hbm, euT, eiT, tailu_hbm, taili_hbm, gu, gi,
                 idxstage, fl_r, fl_b, l1_r, l1_b, l2_r, l2_b, bk_r, bk_b,
                 cur_r, cur_b, tailbuf, outstage, cbuf, sems, osem):
    nc = 2
    wid = lax.axis_index("s") * nc + lax.axis_index("c")

    lo_tc = (wid * NTC) >> 5
    hi_tc = ((wid + 1) * NTC) >> 5
    lo_r = lo_tc * 128
    hi_r = jnp.minimum(hi_tc * 128, NROW)

    iota = lax.iota(jnp.int32, 16)
    neg1 = jnp.full((16,), -1, jnp.int32)

    max_tc = (NROW - CHW) // 128

    def fire(c):
        base = pl.multiple_of(
            jnp.minimum(lo_tc + CTC * c, max_tc) * 128, 128)
        for tab, src in ((0, euT), (1, eiT)):
            for fb in range(4):
                pltpu.async_copy(
                    src.at[pl.ds(fb * 8, 8), pl.ds(base, CHW)],
                    cbuf.at[tab, fb], sems.at[0])

    def wait_chunk():
        for tab in range(2):
            for fb in range(4):
                pltpu.make_async_copy(
                    euT.at[pl.ds(0, 8), pl.ds(0, CHW)],
                    cbuf.at[tab, fb], sems.at[0]).wait()

    pltpu.sync_copy(tailu_hbm, tailbuf.at[0])
    pltpu.sync_copy(taili_hbm, tailbuf.at[1])

    def prefill(k, carry):
        bk_r[pl.ds(k * 16, 16)] = neg1
        return carry

    lax.fori_loop(0, (16 * BCAP) // 16, prefill, 0)

    cnts = []
    for tab, src_hbm in ((0, user_hbm), (1, item_hbm)):
        pltpu.sync_copy(src_hbm, idxstage)

        def bin_step(t, cnt, tab=tab):
            off = t * 16
            bvec = off + iota
            v = idxstage[pl.ds(off, 16)]
            m = (v >= lo_r) & (v < hi_r)
            plsc.store_compressed(fl_r.at[pl.ds(tab * CAP + cnt, 16)], v,
                                  mask=m)
            plsc.store_compressed(fl_b.at[pl.ds(tab * CAP + cnt, 16)], bvec,
                                  mask=m)
            return cnt + _popc(m)

        cnts.append(lax.fori_loop(0, B // 16, bin_step, jnp.int32(0)))

    def split(src_r, src_b, s_base, s_cnt, dst_r, dst_b, d0, d1, mid, nv):
        def body(k, carry):
            cl, cr = carry
            pos = k * 16
            rv = src_r[pl.ds(s_base + pos, 16)]
            bv = src_b[pl.ds(s_base + pos, 16)]
            valid = (pos + iota) < s_cnt
            ml = valid & (rv < mid)
            mr = valid & (rv >= mid)
            plsc.store_compressed(dst_r.at[pl.ds(d0 + cl, 16)], rv, mask=ml)
            plsc.store_compressed(dst_b.at[pl.ds(d0 + cl, 16)], bv, mask=ml)
            plsc.store_compressed(dst_r.at[pl.ds(d1 + cr, 16)], rv, mask=mr)
            plsc.store_compressed(dst_b.at[pl.ds(d1 + cr, 16)], bv, mask=mr)
            return cl + _popc(ml), cr + _popc(mr)

        return lax.fori_loop(0, nv, body, (jnp.int32(0), jnp.int32(0)))

    for tab in range(2):
        c0, c1 = split(fl_r, fl_b, tab * CAP, cnts[tab],
                       l1_r, l1_b, tab * 768, tab * 768 + 384,
                       lo_r + 16384, CAP // 16)
        l2c = []
        for h, ch in ((0, c0), (1, c1)):
            mid = lo_r + h * 16384 + 8192
            a, b_ = split(l1_r, l1_b, tab * 768 + h * 384, ch,
                          l2_r, l2_b, tab * 768 + h * 384,
                          tab * 768 + h * 384 + 192, mid, 384 // 16)
            l2c += [a, b_]
        for q in range(4):
            mid = lo_r + q * 8192 + 4096
            split(l2_r, l2_b, tab * 768 + q * 192, l2c[q],
                  bk_r, bk_b, (tab * 8 + 2 * q) * BCAP,
                  (tab * 8 + 2 * q + 1) * BCAP, mid, 192 // 16)

    fbv_lo = iota >> 3
    fbv_hi = 2 + (iota >> 3)
    subv = iota & 7

    def extract_entries(tab, base, ccnt, from_tail):
        out_ref = gu if tab == 0 else gi

        def entry_vreg(e, carry):
            rv = cur_r[pl.ds(tab * BCAP + e * 16, 16)]
            bv = cur_b[pl.ds(tab * BCAP + e * 16, 16)]
            for k in range(16):
                @pl.when(e * 16 + k < ccnt)
                def _():
                    r = rv[k]
                    bpos = bv[k]
                    n = (tab * OCAP + e * 16 + k) * F
                    if from_tail:
                        rloc = jnp.full((16,), r - TAIL, jnp.int32)
                        lo = plsc.load_gather(tailbuf.at[tab], [iota, rloc])
                        hi = plsc.load_gather(tailbuf.at[tab],
                                              [16 + iota, rloc])
                    else:
                        rlv = jnp.full((16,), r - base, jnp.int32)
                        lo = plsc.load_gather(cbuf.at[tab],
                                              [fbv_lo, subv, rlv])
                        hi = plsc.load_gather(cbuf.at[tab],
                                              [fbv_hi, subv, rlv])
                    outstage[pl.ds(n, 16)] = lo
                    outstage[pl.ds(n + 16, 16)] = hi
                    pltpu.async_copy(
                        outstage.at[pl.ds(n, F)],
                        out_ref.at[pl.ds(pl.multiple_of(bpos * F, F), F)],
                        osem)
            return carry

        lax.fori_loop(0, (ccnt + 15) >> 4, entry_vreg, 0)

    def drain_out(n):
        def body(_, carry):
            pltpu.make_async_copy(outstage.at[pl.ds(0, F)],
                                  gu.at[pl.ds(0, F)], osem).wait()
            return carry
        lax.fori_loop(0, n, body, 0)

    def process(c):
        base = jnp.minimum(lo_tc + CTC * c, max_tc) * 128
        bkt = c >> 3
        fired = jnp.int32(0)
        for tab in range(2):
            boff = (tab * 8) * BCAP
            ccnt = jnp.int32(0)
            for k in range(BCAP // 16):
                pos = k * 16
                rv = bk_r[pl.ds(boff + bkt * BCAP + pos, 16)]
                bv = bk_b[pl.ds(boff + bkt * BCAP + pos, 16)]
                m = (rv >= base) & (rv < base + CHW)
                plsc.store_compressed(
                    cur_r.at[pl.ds(tab * BCAP + ccnt, 16)], rv, mask=m)
                plsc.store_compressed(
                    cur_b.at[pl.ds(tab * BCAP + ccnt, 16)], bv, mask=m)
                ccnt = ccnt + _popc(m)
            ccnt = jnp.minimum(ccnt, OCAP)
            extract_entries(tab, base, ccnt, False)
            fired = fired + ccnt
        drain_out(fired)

    def step(c, carry):
        fire(c)
        wait_chunk()
        process(c)
        return carry

    lax.fori_loop(0, NCH, step, 0)

    for tab in range(2):
        boff = (tab * 8 + 7) * BCAP
        ccnt = jnp.int32(0)
        for k in range(BCAP // 16):
            pos = k * 16
            rv = bk_r[pl.ds(boff + pos, 16)]
            bv = bk_b[pl.ds(boff + pos, 16)]
            m = rv >= TAIL
            plsc.store_compressed(cur_r.at[pl.ds(tab * BCAP + ccnt, 16)], rv,
                                  mask=m)
            plsc.store_compressed(cur_b.at[pl.ds(tab * BCAP + ccnt, 16)], bv,
                                  mask=m)
            ccnt = ccnt + _popc(m)
        ccnt = jnp.minimum(ccnt, OCAP)
        extract_entries(tab, 0, ccnt, True)
        drain_out(ccnt)


def _make_gather_kernel():
    mesh = plsc.VectorSubcoreMesh(core_axis_name="c", subcore_axis_name="s",
                                  num_cores=2, num_subcores=16)
    return pl.kernel(
        _gather_body,
        out_type=(jax.ShapeDtypeStruct((B * F,), jnp.float32),
                  jax.ShapeDtypeStruct((B * F,), jnp.float32)),
        mesh=mesh,
        compiler_params=pltpu.CompilerParams(needs_layout_passes=False),
        scratch_types=[
            pltpu.VMEM((B,), jnp.int32),
            pltpu.VMEM((2 * CAP,), jnp.int32),
            pltpu.VMEM((2 * CAP,), jnp.int32),
            pltpu.VMEM((2 * 768,), jnp.int32),
            pltpu.VMEM((2 * 768,), jnp.int32),
            pltpu.VMEM((2 * 768,), jnp.int32),
            pltpu.VMEM((2 * 768,), jnp.int32),
            pltpu.VMEM((16 * BCAP,), jnp.int32),
            pltpu.VMEM((16 * BCAP,), jnp.int32),
            pltpu.VMEM((2 * BCAP,), jnp.int32),
            pltpu.VMEM((2 * BCAP,), jnp.int32),
            pltpu.VMEM((2, F, 128), jnp.float32),
            pltpu.VMEM((2 * OCAP * F,), jnp.float32),
            pltpu.VMEM((2, 4, 8, CHW), jnp.float32),
            pltpu.SemaphoreType.DMA((1,)),
            pltpu.SemaphoreType.DMA,
        ],
    )


def _compute_body(gu, gi, w_hbm, bias_hbm, out_hbm,
                  ubuf, ibuf, wv, bv, outv, sem_unused):
    nc = 2
    wid = lax.axis_index("s") * nc + lax.axis_index("c")
    bpw = B // NW

    pltpu.sync_copy(gu.at[pl.ds(wid * bpw * F, bpw * F)], ubuf)
    pltpu.sync_copy(gi.at[pl.ds(wid * bpw * F, bpw * F)], ibuf)
    pltpu.sync_copy(w_hbm, wv)
    pltpu.sync_copy(bias_hbm, bv)

    w_lo = wv[pl.ds(0, 16)]
    w_hi = wv[pl.ds(16, 16)]
    wcols = [w_lo[f] if f < 16 else w_hi[f - 16] for f in range(F)]
    bias = bv[...]
    iota32 = lax.iota(jnp.int32, 16) * F

    def group(g, carry):
        idx0 = g * (16 * F) + iota32
        acc = bias
        for f in range(F):
            u = plsc.load_gather(ubuf, [idx0 + f])
            iv = plsc.load_gather(ibuf, [idx0 + f])
            acc = acc + u * iv * wcols[f]
        outv[pl.ds(g * 16, 16)] = acc
        return carry

    lax.fori_loop(0, bpw // 16, group, 0)
    pltpu.sync_copy(outv, out_hbm.at[pl.ds(wid * bpw, bpw)])


def _make_compute_kernel():
    mesh = plsc.VectorSubcoreMesh(core_axis_name="c", subcore_axis_name="s",
                                  num_cores=2, num_subcores=16)
    bpw = B // NW
    return pl.kernel(
        _compute_body,
        out_type=jax.ShapeDtypeStruct((B,), jnp.float32),
        mesh=mesh,
        compiler_params=pltpu.CompilerParams(needs_layout_passes=False),
        scratch_types=[
            pltpu.VMEM((bpw * F,), jnp.float32),
            pltpu.VMEM((bpw * F,), jnp.float32),
            pltpu.VMEM((F,), jnp.float32),
            pltpu.VMEM((16,), jnp.float32),
            pltpu.VMEM((bpw,), jnp.float32),
            pltpu.SemaphoreType.DMA,
        ],
    )


@functools.partial(jax.jit, static_argnames=())
def _gmf(user, item, euT, eiT, tail_u, tail_i, w_vec, bias_vec):
    gu, gi = _make_gather_kernel()(user, item, euT, eiT, tail_u, tail_i)
    return _make_compute_kernel()(gu, gi, w_vec, bias_vec)


def kernel(user, item, embed_user, embed_item, W, b):
    euT = embed_user.T
    eiT = embed_item.T
    tail_u = jnp.pad(euT[:, TAIL:], ((0, 0), (0, 64)))
    tail_i = jnp.pad(eiT[:, TAIL:], ((0, 0), (0, 64)))
    w_vec = W.reshape(F)
    bias_vec = jnp.broadcast_to(b, (16,))
    return _gmf(user, item, euT, eiT, tail_u, tail_i, w_vec, bias_vec)


# speedup vs baseline: 4.5251x; 1.3641x over previous
import functools

import jax
import jax.numpy as jnp
from jax import lax
from jax.experimental import pallas as pl
from jax.experimental.pallas import tpu as pltpu
from jax.experimental.pallas import tpu_sc as plsc

B = 16384
F = 32
NW = 32
NROW = 1000000
NTC = 7813
TAIL = 999936
CTC = 4
NCH = 62
CAP = 768
BCAP = 128
CHW = CTC * 128
OCAP = 64


def _popc(mask):
    return plsc.all_reduce_population_count(mask)[0]


def _gather_body(user_hbm, item_hbm, euT, eiT, tailu_hbm, taili_hbm, gu, gi,
                 idxstage, fl_r, fl_b, l1_r, l1_b, l2_r, l2_b, bk_r, bk_b,
                 cur_r, cur_b, tailbuf, outstage, cbuf, sems, osem):
    nc = 2
    wid = lax.axis_index("s") * nc + lax.axis_index("c")

    lo_tc = (wid * NTC) >> 5
    hi_tc = ((wid + 1) * NTC) >> 5
    lo_r = lo_tc * 128
    hi_r = jnp.minimum(hi_tc * 128, NROW)

    iota = lax.iota(jnp.int32, 16)
    neg1 = jnp.full((16,), -1, jnp.int32)

    max_tc = (NROW - CHW) // 128

    def fire(c, slot):
        base = pl.multiple_of(
            jnp.minimum(lo_tc + CTC * c, max_tc) * 128, 128)
        for tab, src in ((0, euT), (1, eiT)):
            for fb in range(4):
                pltpu.async_copy(
                    src.at[pl.ds(fb * 8, 8), pl.ds(base, CHW)],
                    cbuf.at[slot, tab, fb], sems.at[slot])

    def wait_chunk(slot):
        for tab in range(2):
            for fb in range(4):
                pltpu.make_async_copy(
                    euT.at[pl.ds(0, 8), pl.ds(0, CHW)],
                    cbuf.at[slot, tab, fb], sems.at[slot]).wait()

    pltpu.sync_copy(tailu_hbm, tailbuf.at[0])
    pltpu.sync_copy(taili_hbm, tailbuf.at[1])

    def prefill(k, carry):
        bk_r[pl.ds(k * 16, 16)] = neg1
        return carry

    lax.fori_loop(0, (16 * BCAP) // 16, prefill, 0)

    cnts = []
    for tab, src_hbm in ((0, user_hbm), (1, item_hbm)):
        pltpu.sync_copy(src_hbm, idxstage)

        def bin_step(t, cnt, tab=tab):
            off = t * 16
            bvec = off + iota
            v = idxstage[pl.ds(off, 16)]
            m = (v >= lo_r) & (v < hi_r)
            plsc.store_compressed(fl_r.at[pl.ds(tab * CAP + cnt, 16)], v,
                                  mask=m)
            plsc.store_compressed(fl_b.at[pl.ds(tab * CAP + cnt, 16)], bvec,
                                  mask=m)
            return cnt + _popc(m)

        cnts.append(lax.fori_loop(0, B // 16, bin_step, jnp.int32(0)))

    def split(src_r, src_b, s_base, s_cnt, dst_r, dst_b, d0, d1, mid, nv):
        def body(k, carry):
            cl, cr = carry
            pos = k * 16
            rv = src_r[pl.ds(s_base + pos, 16)]
            bv = src_b[pl.ds(s_base + pos, 16)]
            valid = (pos + iota) < s_cnt
            ml = valid & (rv < mid)
            mr = valid & (rv >= mid)
            plsc.store_compressed(dst_r.at[pl.ds(d0 + cl, 16)], rv, mask=ml)
            plsc.store_compressed(dst_b.at[pl.ds(d0 + cl, 16)], bv, mask=ml)
            plsc.store_compressed(dst_r.at[pl.ds(d1 + cr, 16)], rv, mask=mr)
            plsc.store_compressed(dst_b.at[pl.ds(d1 + cr, 16)], bv, mask=mr)
            return cl + _popc(ml), cr + _popc(mr)

        return lax.fori_loop(0, nv, body, (jnp.int32(0), jnp.int32(0)))

    for tab in range(2):
        c0, c1 = split(fl_r, fl_b, tab * CAP, cnts[tab],
                       l1_r, l1_b, tab * 768, tab * 768 + 384,
                       lo_r + 16384, CAP // 16)
        l2c = []
        for h, ch in ((0, c0), (1, c1)):
            mid = lo_r + h * 16384 + 8192
            a, b_ = split(l1_r, l1_b, tab * 768 + h * 384, ch,
                          l2_r, l2_b, tab * 768 + h * 384,
                          tab * 768 + h * 384 + 192, mid, 384 // 16)
            l2c += [a, b_]
        for q in range(4):
            mid = lo_r + q * 8192 + 4096
            split(l2_r, l2_b, tab * 768 + q * 192, l2c[q],
                  bk_r, bk_b, (tab * 8 + 2 * q) * BCAP,
                  (tab * 8 + 2 * q + 1) * BCAP, mid, 192 // 16)

    fbv_lo = iota >> 3
    fbv_hi = 2 + (iota >> 3)
    subv = iota & 7

    def extract_entries(tab, base, ccnt, from_tail, slot=0):
        out_ref = gu if tab == 0 else gi

        def entry_vreg(e, carry):
            rv = cur_r[pl.ds(tab * BCAP + e * 16, 16)]
            bv = cur_b[pl.ds(tab * BCAP + e * 16, 16)]
            for k in range(16):
                @pl.when(e * 16 + k < ccnt)
                def _():
                    r = rv[k]
                    bpos = bv[k]
                    n = (tab * OCAP + e * 16 + k) * F
                    if from_tail:
                        rloc = jnp.full((16,), r - TAIL, jnp.int32)
                        lo = plsc.load_gather(tailbuf.at[tab], [iota, rloc])
                        hi = plsc.load_gather(tailbuf.at[tab],
                                              [16 + iota, rloc])
                    else:
                        rlv = jnp.full((16,), r - base, jnp.int32)
                        lo = plsc.load_gather(cbuf.at[slot, tab],
                                              [fbv_lo, subv, rlv])
                        hi = plsc.load_gather(cbuf.at[slot, tab],
                                              [fbv_hi, subv, rlv])
                    outstage[pl.ds(n, 16)] = lo
                    outstage[pl.ds(n + 16, 16)] = hi
                    pltpu.async_copy(
                        outstage.at[pl.ds(n, F)],
                        out_ref.at[pl.ds(pl.multiple_of(bpos * F, F), F)],
                        osem)
            return carry

        lax.fori_loop(0, (ccnt + 15) >> 4, entry_vreg, 0)

    def drain_out(n):
        def body(_, carry):
            pltpu.make_async_copy(outstage.at[pl.ds(0, F)],
                                  gu.at[pl.ds(0, F)], osem).wait()
            return carry
        lax.fori_loop(0, n, body, 0)

    def process(c, slot):
        base = jnp.minimum(lo_tc + CTC * c, max_tc) * 128
        bkt = c >> 3
        fired = jnp.int32(0)
        for tab in range(2):
            boff = (tab * 8) * BCAP
            ccnt = jnp.int32(0)
            for k in range(BCAP // 16):
                pos = k * 16
                rv = bk_r[pl.ds(boff + bkt * BCAP + pos, 16)]
                bv = bk_b[pl.ds(boff + bkt * BCAP + pos, 16)]
                m = (rv >= base) & (rv < base + CHW)
                plsc.store_compressed(
                    cur_r.at[pl.ds(tab * BCAP + ccnt, 16)], rv, mask=m)
                plsc.store_compressed(
                    cur_b.at[pl.ds(tab * BCAP + ccnt, 16)], bv, mask=m)
                ccnt = ccnt + _popc(m)
            ccnt = jnp.minimum(ccnt, OCAP)
            extract_entries(tab, base, ccnt, False, slot)
            fired = fired + ccnt
        drain_out(fired)

    fire(0, 0)

    def pair(t, carry):
        fire(2 * t + 1, 1)
        wait_chunk(0)
        process(2 * t, 0)

        @pl.when(2 * t + 2 < NCH)
        def _():
            fire(2 * t + 2, 0)

        wait_chunk(1)
        process(2 * t + 1, 1)
        return carry

    lax.fori_loop(0, NCH // 2, pair, 0)

    for tab in range(2):
        boff = (tab * 8 + 7) * BCAP
        ccnt = jnp.int32(0)
        for k in range(BCAP // 16):
            pos = k * 16
            rv = bk_r[pl.ds(boff + pos, 16)]
            bv = bk_b[pl.ds(boff + pos, 16)]
            m = rv >= TAIL
            plsc.store_compressed(cur_r.at[pl.ds(tab * BCAP + ccnt, 16)], rv,
                                  mask=m)
            plsc.store_compressed(cur_b.at[pl.ds(tab * BCAP + ccnt, 16)], bv,
                                  mask=m)
            ccnt = ccnt + _popc(m)
        ccnt = jnp.minimum(ccnt, OCAP)
        extract_entries(tab, 0, ccnt, True)
        drain_out(ccnt)


def _make_gather_kernel():
    mesh = plsc.VectorSubcoreMesh(core_axis_name="c", subcore_axis_name="s",
                                  num_cores=2, num_subcores=16)
    return pl.kernel(
        _gather_body,
        out_type=(jax.ShapeDtypeStruct((B * F,), jnp.float32),
                  jax.ShapeDtypeStruct((B * F,), jnp.float32)),
        mesh=mesh,
        compiler_params=pltpu.CompilerParams(needs_layout_passes=False),
        scratch_types=[
            pltpu.VMEM((B,), jnp.int32),
            pltpu.VMEM((2 * CAP,), jnp.int32),
            pltpu.VMEM((2 * CAP,), jnp.int32),
            pltpu.VMEM((2 * 768,), jnp.int32),
            pltpu.VMEM((2 * 768,), jnp.int32),
            pltpu.VMEM((2 * 768,), jnp.int32),
            pltpu.VMEM((2 * 768,), jnp.int32),
            pltpu.VMEM((16 * BCAP,), jnp.int32),
            pltpu.VMEM((16 * BCAP,), jnp.int32),
            pltpu.VMEM((2 * BCAP,), jnp.int32),
            pltpu.VMEM((2 * BCAP,), jnp.int32),
            pltpu.VMEM((2, F, 128), jnp.float32),
            pltpu.VMEM((2 * OCAP * F,), jnp.float32),
            pltpu.VMEM((2, 2, 4, 8, CHW), jnp.float32),
            pltpu.SemaphoreType.DMA((2,)),
            pltpu.SemaphoreType.DMA,
        ],
    )


def _compute_body(gu, gi, w_hbm, bias_hbm, out_hbm,
                  ubuf, ibuf, wv, bv, outv, sem_unused):
    nc = 2
    wid = lax.axis_index("s") * nc + lax.axis_index("c")
    bpw = B // NW

    pltpu.sync_copy(gu.at[pl.ds(wid * bpw * F, bpw * F)], ubuf)
    pltpu.sync_copy(gi.at[pl.ds(wid * bpw * F, bpw * F)], ibuf)
    pltpu.sync_copy(w_hbm, wv)
    pltpu.sync_copy(bias_hbm, bv)

    w_lo = wv[pl.ds(0, 16)]
    w_hi = wv[pl.ds(16, 16)]
    wcols = [w_lo[f] if f < 16 else w_hi[f - 16] for f in range(F)]
    bias = bv[...]
    iota32 = lax.iota(jnp.int32, 16) * F

    def group(g, carry):
        idx0 = g * (16 * F) + iota32
        acc = bias
        for f in range(F):
            u = plsc.load_gather(ubuf, [idx0 + f])
            iv = plsc.load_gather(ibuf, [idx0 + f])
            acc = acc + u * iv * wcols[f]
        outv[pl.ds(g * 16, 16)] = acc
        return carry

    lax.fori_loop(0, bpw // 16, group, 0)
    pltpu.sync_copy(outv, out_hbm.at[pl.ds(wid * bpw, bpw)])


def _make_compute_kernel():
    mesh = plsc.VectorSubcoreMesh(core_axis_name="c", subcore_axis_name="s",
                                  num_cores=2, num_subcores=16)
    bpw = B // NW
    return pl.kernel(
        _compute_body,
        out_type=jax.ShapeDtypeStruct((B,), jnp.float32),
        mesh=mesh,
        compiler_params=pltpu.CompilerParams(needs_layout_passes=False),
        scratch_types=[
            pltpu.VMEM((bpw * F,), jnp.float32),
            pltpu.VMEM((bpw * F,), jnp.float32),
            pltpu.VMEM((F,), jnp.float32),
            pltpu.VMEM((16,), jnp.float32),
            pltpu.VMEM((bpw,), jnp.float32),
            pltpu.SemaphoreType.DMA,
        ],
    )


@functools.partial(jax.jit, static_argnames=())
def _gmf(user, item, euT, eiT, tail_u, tail_i, w_vec, bias_vec):
    gu, gi = _make_gather_kernel()(user, item, euT, eiT, tail_u, tail_i)
    return _make_compute_kernel()(gu, gi, w_vec, bias_vec)


def kernel(user, item, embed_user, embed_item, W, b):
    euT = embed_user.T
    eiT = embed_item.T
    tail_u = jnp.pad(euT[:, TAIL:], ((0, 0), (0, 64)))
    tail_i = jnp.pad(eiT[:, TAIL:], ((0, 0), (0, 64)))
    w_vec = W.reshape(F)
    bias_vec = jnp.broadcast_to(b, (16,))
    return _gmf(user, item, euT, eiT, tail_u, tail_i, w_vec, bias_vec)


# speedup vs baseline: 4.5354x; 1.0023x over previous
import functools

import jax
import jax.numpy as jnp
from jax import lax
from jax.experimental import pallas as pl
from jax.experimental.pallas import tpu as pltpu
from jax.experimental.pallas import tpu_sc as plsc

B = 16384
F = 32
NW = 32
NROW = 1000000
NTC = 7813
TAIL = 999936
CTC = 4
NCH = 62
CAP = 768
BCAP = 128
CHW = CTC * 128
OCAP = 64


def _popc(mask):
    return plsc.all_reduce_population_count(mask)[0]


def _gather_body(user_hbm, item_hbm, euT, eiT, tailu_hbm, taili_hbm, gu, gi,
                 idxstage, fl_r, fl_b, l1_r, l1_b, l2_r, l2_b, bk_r, bk_b,
                 cur_r, cur_b, tailbuf, outstage, cbuf, sems, osem):
    nc = 2
    wid = lax.axis_index("s") * nc + lax.axis_index("c")

    lo_tc = (wid * NTC) >> 5
    hi_tc = ((wid + 1) * NTC) >> 5
    lo_r = lo_tc * 128
    hi_r = jnp.minimum(hi_tc * 128, NROW)

    iota = lax.iota(jnp.int32, 16)
    neg1 = jnp.full((16,), -1, jnp.int32)

    max_tc = (NROW - CHW) // 128

    def fire(c, slot):
        base = pl.multiple_of(
            jnp.minimum(lo_tc + CTC * c, max_tc) * 128, 128)
        for tab, src in ((0, euT), (1, eiT)):
            for fb in range(4):
                pltpu.async_copy(
                    src.at[pl.ds(fb * 8, 8), pl.ds(base, CHW)],
                    cbuf.at[slot, tab, fb], sems.at[slot])

    def wait_chunk(slot):
        for tab in range(2):
            for fb in range(4):
                pltpu.make_async_copy(
                    euT.at[pl.ds(0, 8), pl.ds(0, CHW)],
                    cbuf.at[slot, tab, fb], sems.at[slot]).wait()

    pltpu.sync_copy(tailu_hbm, tailbuf.at[0])
    pltpu.sync_copy(taili_hbm, tailbuf.at[1])

    def prefill(k, carry):
        bk_r[pl.ds(k * 16, 16)] = neg1
        return carry

    lax.fori_loop(0, (16 * BCAP) // 16, prefill, 0)

    cnts = []
    for tab, src_hbm in ((0, user_hbm), (1, item_hbm)):
        pltpu.sync_copy(src_hbm, idxstage)

        def bin_step(t, cnt, tab=tab):
            for u in range(4):
                off = t * 64 + u * 16
                bvec = off + iota
                v = idxstage[pl.ds(off, 16)]
                m = (v >= lo_r) & (v < hi_r)
                plsc.store_compressed(fl_r.at[pl.ds(tab * CAP + cnt, 16)], v,
                                      mask=m)
                plsc.store_compressed(fl_b.at[pl.ds(tab * CAP + cnt, 16)],
                                      bvec, mask=m)
                cnt = cnt + _popc(m)
            return cnt

        cnts.append(lax.fori_loop(0, B // 64, bin_step, jnp.int32(0)))

    def split(src_r, src_b, s_base, s_cnt, dst_r, dst_b, d0, d1, mid, nv):
        def body(k, carry):
            cl, cr = carry
            pos = k * 16
            rv = src_r[pl.ds(s_base + pos, 16)]
            bv = src_b[pl.ds(s_base + pos, 16)]
            valid = (pos + iota) < s_cnt
            ml = valid & (rv < mid)
            mr = valid & (rv >= mid)
            plsc.store_compressed(dst_r.at[pl.ds(d0 + cl, 16)], rv, mask=ml)
            plsc.store_compressed(dst_b.at[pl.ds(d0 + cl, 16)], bv, mask=ml)
            plsc.store_compressed(dst_r.at[pl.ds(d1 + cr, 16)], rv, mask=mr)
            plsc.store_compressed(dst_b.at[pl.ds(d1 + cr, 16)], bv, mask=mr)
            return cl + _popc(ml), cr + _popc(mr)

        return lax.fori_loop(0, nv, body, (jnp.int32(0), jnp.int32(0)))

    for tab in range(2):
        c0, c1 = split(fl_r, fl_b, tab * CAP, cnts[tab],
                       l1_r, l1_b, tab * 768, tab * 768 + 384,
                       lo_r + 16384, CAP // 16)
        l2c = []
        for h, ch in ((0, c0), (1, c1)):
            mid = lo_r + h * 16384 + 8192
            a, b_ = split(l1_r, l1_b, tab * 768 + h * 384, ch,
                          l2_r, l2_b, tab * 768 + h * 384,
                          tab * 768 + h * 384 + 192, mid, 384 // 16)
            l2c += [a, b_]
        for q in range(4):
            mid = lo_r + q * 8192 + 4096
            split(l2_r, l2_b, tab * 768 + q * 192, l2c[q],
                  bk_r, bk_b, (tab * 8 + 2 * q) * BCAP,
                  (tab * 8 + 2 * q + 1) * BCAP, mid, 192 // 16)

    fbv_lo = iota >> 3
    fbv_hi = 2 + (iota >> 3)
    subv = iota & 7

    def extract_entries(tab, base, ccnt, from_tail, slot=0):
        out_ref = gu if tab == 0 else gi

        def entry_vreg(e, carry):
            rv = cur_r[pl.ds(tab * BCAP + e * 16, 16)]
            bv = cur_b[pl.ds(tab * BCAP + e * 16, 16)]
            for k in range(16):
                @pl.when(e * 16 + k < ccnt)
                def _():
                    r = rv[k]
                    bpos = bv[k]
                    n = (tab * OCAP + e * 16 + k) * F
                    if from_tail:
                        rloc = jnp.full((16,), r - TAIL, jnp.int32)
                        lo = plsc.load_gather(tailbuf.at[tab], [iota, rloc])
                        hi = plsc.load_gather(tailbuf.at[tab],
                                              [16 + iota, rloc])
                    else:
                        rlv = jnp.full((16,), r - base, jnp.int32)
                        lo = plsc.load_gather(cbuf.at[slot, tab],
                                              [fbv_lo, subv, rlv])
                        hi = plsc.load_gather(cbuf.at[slot, tab],
                                              [fbv_hi, subv, rlv])
                    outstage[pl.ds(n, 16)] = lo
                    outstage[pl.ds(n + 16, 16)] = hi
                    pltpu.async_copy(
                        outstage.at[pl.ds(n, F)],
                        out_ref.at[pl.ds(pl.multiple_of(bpos * F, F), F)],
                        osem)
            return carry

        lax.fori_loop(0, (ccnt + 15) >> 4, entry_vreg, 0)

    def drain_out(n):
        def body(_, carry):
            pltpu.make_async_copy(outstage.at[pl.ds(0, F)],
                                  gu.at[pl.ds(0, F)], osem).wait()
            return carry
        lax.fori_loop(0, n, body, 0)

    def process(c, slot):
        base = jnp.minimum(lo_tc + CTC * c, max_tc) * 128
        bkt = c >> 3
        fired = jnp.int32(0)
        for tab in range(2):
            boff = (tab * 8) * BCAP
            ccnt = jnp.int32(0)
            for k in range(BCAP // 16):
                pos = k * 16
                rv = bk_r[pl.ds(boff + bkt * BCAP + pos, 16)]
                bv = bk_b[pl.ds(boff + bkt * BCAP + pos, 16)]
                m = (rv >= base) & (rv < base + CHW)
                plsc.store_compressed(
                    cur_r.at[pl.ds(tab * BCAP + ccnt, 16)], rv, mask=m)
                plsc.store_compressed(
                    cur_b.at[pl.ds(tab * BCAP + ccnt, 16)], bv, mask=m)
                ccnt = ccnt + _popc(m)
            ccnt = jnp.minimum(ccnt, OCAP)
            extract_entries(tab, base, ccnt, False, slot)
            fired = fired + ccnt
        drain_out(fired)

    fire(0, 0)

    def pair(t, carry):
        fire(2 * t + 1, 1)
        wait_chunk(0)
        process(2 * t, 0)

        @pl.when(2 * t + 2 < NCH)
        def _():
            fire(2 * t + 2, 0)

        wait_chunk(1)
        process(2 * t + 1, 1)
        return carry

    lax.fori_loop(0, NCH // 2, pair, 0)

    for tab in range(2):
        boff = (tab * 8 + 7) * BCAP
        ccnt = jnp.int32(0)
        for k in range(BCAP // 16):
            pos = k * 16
            rv = bk_r[pl.ds(boff + pos, 16)]
            bv = bk_b[pl.ds(boff + pos, 16)]
            m = rv >= TAIL
            plsc.store_compressed(cur_r.at[pl.ds(tab * BCAP + ccnt, 16)], rv,
                                  mask=m)
            plsc.store_compressed(cur_b.at[pl.ds(tab * BCAP + ccnt, 16)], bv,
                                  mask=m)
            ccnt = ccnt + _popc(m)
        ccnt = jnp.minimum(ccnt, OCAP)
        extract_entries(tab, 0, ccnt, True)
        drain_out(ccnt)


def _make_gather_kernel():
    mesh = plsc.VectorSubcoreMesh(core_axis_name="c", subcore_axis_name="s",
                                  num_cores=2, num_subcores=16)
    return pl.kernel(
        _gather_body,
        out_type=(jax.ShapeDtypeStruct((B * F,), jnp.float32),
                  jax.ShapeDtypeStruct((B * F,), jnp.float32)),
        mesh=mesh,
        compiler_params=pltpu.CompilerParams(needs_layout_passes=False),
        scratch_types=[
            pltpu.VMEM((B,), jnp.int32),
            pltpu.VMEM((2 * CAP,), jnp.int32),
            pltpu.VMEM((2 * CAP,), jnp.int32),
            pltpu.VMEM((2 * 768,), jnp.int32),
            pltpu.VMEM((2 * 768,), jnp.int32),
            pltpu.VMEM((2 * 768,), jnp.int32),
            pltpu.VMEM((2 * 768,), jnp.int32),
            pltpu.VMEM((16 * BCAP,), jnp.int32),
            pltpu.VMEM((16 * BCAP,), jnp.int32),
            pltpu.VMEM((2 * BCAP,), jnp.int32),
            pltpu.VMEM((2 * BCAP,), jnp.int32),
            pltpu.VMEM((2, F, 128), jnp.float32),
            pltpu.VMEM((2 * OCAP * F,), jnp.float32),
            pltpu.VMEM((2, 2, 4, 8, CHW), jnp.float32),
            pltpu.SemaphoreType.DMA((2,)),
            pltpu.SemaphoreType.DMA,
        ],
    )


def _compute_body(gu, gi, w_hbm, bias_hbm, out_hbm,
                  ubuf, ibuf, wv, bv, outv, sem_unused):
    nc = 2
    wid = lax.axis_index("s") * nc + lax.axis_index("c")
    bpw = B // NW

    pltpu.sync_copy(gu.at[pl.ds(wid * bpw * F, bpw * F)], ubuf)
    pltpu.sync_copy(gi.at[pl.ds(wid * bpw * F, bpw * F)], ibuf)
    pltpu.sync_copy(w_hbm, wv)
    pltpu.sync_copy(bias_hbm, bv)

    w_lo = wv[pl.ds(0, 16)]
    w_hi = wv[pl.ds(16, 16)]
    wcols = [w_lo[f] if f < 16 else w_hi[f - 16] for f in range(F)]
    bias = bv[...]
    iota32 = lax.iota(jnp.int32, 16) * F

    def group(g, carry):
        idx0 = g * (16 * F) + iota32
        acc = bias
        for f in range(F):
            u = plsc.load_gather(ubuf, [idx0 + f])
            iv = plsc.load_gather(ibuf, [idx0 + f])
            acc = acc + u * iv * wcols[f]
        outv[pl.ds(g * 16, 16)] = acc
        return carry

    lax.fori_loop(0, bpw // 16, group, 0)
    pltpu.sync_copy(outv, out_hbm.at[pl.ds(wid * bpw, bpw)])


def _make_compute_kernel():
    mesh = plsc.VectorSubcoreMesh(core_axis_name="c", subcore_axis_name="s",
                                  num_cores=2, num_subcores=16)
    bpw = B // NW
    return pl.kernel(
        _compute_body,
        out_type=jax.ShapeDtypeStruct((B,), jnp.float32),
        mesh=mesh,
        compiler_params=pltpu.CompilerParams(needs_layout_passes=False),
        scratch_types=[
            pltpu.VMEM((bpw * F,), jnp.float32),
            pltpu.VMEM((bpw * F,), jnp.float32),
            pltpu.VMEM((F,), jnp.float32),
            pltpu.VMEM((16,), jnp.float32),
            pltpu.VMEM((bpw,), jnp.float32),
            pltpu.SemaphoreType.DMA,
        ],
    )


@functools.partial(jax.jit, static_argnames=())
def _gmf(user, item, euT, eiT, tail_u, tail_i, w_vec, bias_vec):
    gu, gi = _make_gather_kernel()(user, item, euT, eiT, tail_u, tail_i)
    return _make_compute_kernel()(gu, gi, w_vec, bias_vec)


def kernel(user, item, embed_user, embed_item, W, b):
    euT = embed_user.T
    eiT = embed_item.T
    tail_u = jnp.pad(euT[:, TAIL:], ((0, 0), (0, 64)))
    tail_i = jnp.pad(eiT[:, TAIL:], ((0, 0), (0, 64)))
    w_vec = W.reshape(F)
    bias_vec = jnp.broadcast_to(b, (16,))
    return _gmf(user, item, euT, eiT, tail_u, tail_i, w_vec, bias_vec)


# speedup vs baseline: 4.5400x; 1.0010x over previous
import functools

import jax
import jax.numpy as jnp
from jax import lax
from jax.experimental import pallas as pl
from jax.experimental.pallas import tpu as pltpu
from jax.experimental.pallas import tpu_sc as plsc

B = 16384
F = 32
NW = 32
NROW = 1000000
NTC = 7813
TAIL = 999936
CTC = 4
NCH = 62
CAP = 768
BCAP = 128
CHW = CTC * 128
OCAP = 64


def _popc(mask):
    return plsc.all_reduce_population_count(mask)[0]


def _gather_body(user_hbm, item_hbm, euT, eiT, tailu_hbm, taili_hbm, gu, gi,
                 idxstage, fl_r, fl_b, l1_r, l1_b, l2_r, l2_b, bk_r, bk_b,
                 cur_r, cur_b, tailbuf, outstage, cbuf, sems, osem):
    nc = 2
    wid = lax.axis_index("s") * nc + lax.axis_index("c")

    lo_tc = (wid * NTC) >> 5
    hi_tc = ((wid + 1) * NTC) >> 5
    lo_r = lo_tc * 128
    hi_r = jnp.minimum(hi_tc * 128, NROW)

    iota = lax.iota(jnp.int32, 16)
    neg1 = jnp.full((16,), -1, jnp.int32)

    max_tc = (NROW - CHW) // 128

    def fire(c, slot):
        base = pl.multiple_of(
            jnp.minimum(lo_tc + CTC * c, max_tc) * 128, 128)
        for tab, src in ((0, euT), (1, eiT)):
            pltpu.async_copy(
                src.at[:, pl.ds(base, CHW)],
                cbuf.at[slot, tab], sems.at[slot])

    def wait_chunk(slot):
        for tab in range(2):
            pltpu.make_async_copy(
                euT.at[:, pl.ds(0, CHW)],
                cbuf.at[slot, tab], sems.at[slot]).wait()

    pltpu.sync_copy(tailu_hbm, tailbuf.at[0])
    pltpu.sync_copy(taili_hbm, tailbuf.at[1])

    def prefill(k, carry):
        bk_r[pl.ds(k * 16, 16)] = neg1
        return carry

    lax.fori_loop(0, (16 * BCAP) // 16, prefill, 0)

    cnts = []
    for tab, src_hbm in ((0, user_hbm), (1, item_hbm)):
        pltpu.sync_copy(src_hbm, idxstage)

        def bin_step(t, cnt, tab=tab):
            for u in range(4):
                off = t * 64 + u * 16
                bvec = off + iota
                v = idxstage[pl.ds(off, 16)]
                m = (v >= lo_r) & (v < hi_r)
                plsc.store_compressed(fl_r.at[pl.ds(tab * CAP + cnt, 16)], v,
                                      mask=m)
                plsc.store_compressed(fl_b.at[pl.ds(tab * CAP + cnt, 16)],
                                      bvec, mask=m)
                cnt = cnt + _popc(m)
            return cnt

        cnts.append(lax.fori_loop(0, B // 64, bin_step, jnp.int32(0)))

    def split(src_r, src_b, s_base, s_cnt, dst_r, dst_b, d0, d1, mid, nv):
        def body(k, carry):
            cl, cr = carry
            pos = k * 16
            rv = src_r[pl.ds(s_base + pos, 16)]
            bv = src_b[pl.ds(s_base + pos, 16)]
            valid = (pos + iota) < s_cnt
            ml = valid & (rv < mid)
            mr = valid & (rv >= mid)
            plsc.store_compressed(dst_r.at[pl.ds(d0 + cl, 16)], rv, mask=ml)
            plsc.store_compressed(dst_b.at[pl.ds(d0 + cl, 16)], bv, mask=ml)
            plsc.store_compressed(dst_r.at[pl.ds(d1 + cr, 16)], rv, mask=mr)
            plsc.store_compressed(dst_b.at[pl.ds(d1 + cr, 16)], bv, mask=mr)
            return cl + _popc(ml), cr + _popc(mr)

        return lax.fori_loop(0, nv, body, (jnp.int32(0), jnp.int32(0)))

    for tab in range(2):
        c0, c1 = split(fl_r, fl_b, tab * CAP, cnts[tab],
                       l1_r, l1_b, tab * 768, tab * 768 + 384,
                       lo_r + 16384, CAP // 16)
        l2c = []
        for h, ch in ((0, c0), (1, c1)):
            mid = lo_r + h * 16384 + 8192
            a, b_ = split(l1_r, l1_b, tab * 768 + h * 384, ch,
                          l2_r, l2_b, tab * 768 + h * 384,
                          tab * 768 + h * 384 + 192, mid, 384 // 16)
            l2c += [a, b_]
        for q in range(4):
            mid = lo_r + q * 8192 + 4096
            split(l2_r, l2_b, tab * 768 + q * 192, l2c[q],
                  bk_r, bk_b, (tab * 8 + 2 * q) * BCAP,
                  (tab * 8 + 2 * q + 1) * BCAP, mid, 192 // 16)

    fbv_lo = iota >> 3
    fbv_hi = 2 + (iota >> 3)
    subv = iota & 7

    def extract_entries(tab, base, ccnt, from_tail, slot=0):
        out_ref = gu if tab == 0 else gi

        def entry_vreg(e, carry):
            rv = cur_r[pl.ds(tab * BCAP + e * 16, 16)]
            bv = cur_b[pl.ds(tab * BCAP + e * 16, 16)]
            for k in range(16):
                @pl.when(e * 16 + k < ccnt)
                def _():
                    r = rv[k]
                    bpos = bv[k]
                    n = (tab * OCAP + e * 16 + k) * F
                    if from_tail:
                        rloc = jnp.full((16,), r - TAIL, jnp.int32)
                        lo = plsc.load_gather(tailbuf.at[tab], [iota, rloc])
                        hi = plsc.load_gather(tailbuf.at[tab],
                                              [16 + iota, rloc])
                    else:
                        rlv = jnp.full((16,), r - base, jnp.int32)
                        lo = plsc.load_gather(cbuf.at[slot, tab],
                                              [iota, rlv])
                        hi = plsc.load_gather(cbuf.at[slot, tab],
                                              [16 + iota, rlv])
                    outstage[pl.ds(n, 16)] = lo
                    outstage[pl.ds(n + 16, 16)] = hi
                    pltpu.async_copy(
                        outstage.at[pl.ds(n, F)],
                        out_ref.at[pl.ds(pl.multiple_of(bpos * F, F), F)],
                        osem)
            return carry

        lax.fori_loop(0, (ccnt + 15) >> 4, entry_vreg, 0)

    def drain_out(n):
        def body(_, carry):
            pltpu.make_async_copy(outstage.at[pl.ds(0, F)],
                                  gu.at[pl.ds(0, F)], osem).wait()
            return carry
        lax.fori_loop(0, n, body, 0)

    def process(c, slot):
        base = jnp.minimum(lo_tc + CTC * c, max_tc) * 128
        bkt = c >> 3
        fired = jnp.int32(0)
        for tab in range(2):
            boff = (tab * 8) * BCAP
            ccnt = jnp.int32(0)
            for k in range(BCAP // 16):
                pos = k * 16
                rv = bk_r[pl.ds(boff + bkt * BCAP + pos, 16)]
                bv = bk_b[pl.ds(boff + bkt * BCAP + pos, 16)]
                m = (rv >= base) & (rv < base + CHW)
                plsc.store_compressed(
                    cur_r.at[pl.ds(tab * BCAP + ccnt, 16)], rv, mask=m)
                plsc.store_compressed(
                    cur_b.at[pl.ds(tab * BCAP + ccnt, 16)], bv, mask=m)
                ccnt = ccnt + _popc(m)
            ccnt = jnp.minimum(ccnt, OCAP)
            extract_entries(tab, base, ccnt, False, slot)
            fired = fired + ccnt
        drain_out(fired)

    fire(0, 0)

    def pair(t, carry):
        fire(2 * t + 1, 1)
        wait_chunk(0)
        process(2 * t, 0)

        @pl.when(2 * t + 2 < NCH)
        def _():
            fire(2 * t + 2, 0)

        wait_chunk(1)
        process(2 * t + 1, 1)
        return carry

    lax.fori_loop(0, NCH // 2, pair, 0)

    for tab in range(2):
        boff = (tab * 8 + 7) * BCAP
        ccnt = jnp.int32(0)
        for k in range(BCAP // 16):
            pos = k * 16
            rv = bk_r[pl.ds(boff + pos, 16)]
            bv = bk_b[pl.ds(boff + pos, 16)]
            m = rv >= TAIL
            plsc.store_compressed(cur_r.at[pl.ds(tab * BCAP + ccnt, 16)], rv,
                                  mask=m)
            plsc.store_compressed(cur_b.at[pl.ds(tab * BCAP + ccnt, 16)], bv,
                                  mask=m)
            ccnt = ccnt + _popc(m)
        ccnt = jnp.minimum(ccnt, OCAP)
        extract_entries(tab, 0, ccnt, True)
        drain_out(ccnt)


def _make_gather_kernel():
    mesh = plsc.VectorSubcoreMesh(core_axis_name="c", subcore_axis_name="s",
                                  num_cores=2, num_subcores=16)
    return pl.kernel(
        _gather_body,
        out_type=(jax.ShapeDtypeStruct((B * F,), jnp.float32),
                  jax.ShapeDtypeStruct((B * F,), jnp.float32)),
        mesh=mesh,
        compiler_params=pltpu.CompilerParams(needs_layout_passes=False),
        scratch_types=[
            pltpu.VMEM((B,), jnp.int32),
            pltpu.VMEM((2 * CAP,), jnp.int32),
            pltpu.VMEM((2 * CAP,), jnp.int32),
            pltpu.VMEM((2 * 768,), jnp.int32),
            pltpu.VMEM((2 * 768,), jnp.int32),
            pltpu.VMEM((2 * 768,), jnp.int32),
            pltpu.VMEM((2 * 768,), jnp.int32),
            pltpu.VMEM((16 * BCAP,), jnp.int32),
            pltpu.VMEM((16 * BCAP,), jnp.int32),
            pltpu.VMEM((2 * BCAP,), jnp.int32),
            pltpu.VMEM((2 * BCAP,), jnp.int32),
            pltpu.VMEM((2, F, 128), jnp.float32),
            pltpu.VMEM((2 * OCAP * F,), jnp.float32),
            pltpu.VMEM((2, 2, F, CHW), jnp.float32),
            pltpu.SemaphoreType.DMA((2,)),
            pltpu.SemaphoreType.DMA,
        ],
    )


def _compute_body(gu, gi, w_hbm, bias_hbm, out_hbm,
                  ubuf, ibuf, wv, bv, outv, sem_unused):
    nc = 2
    wid = lax.axis_index("s") * nc + lax.axis_index("c")
    bpw = B // NW

    pltpu.sync_copy(gu.at[pl.ds(wid * bpw * F, bpw * F)], ubuf)
    pltpu.sync_copy(gi.at[pl.ds(wid * bpw * F, bpw * F)], ibuf)
    pltpu.sync_copy(w_hbm, wv)
    pltpu.sync_copy(bias_hbm, bv)

    w_lo = wv[pl.ds(0, 16)]
    w_hi = wv[pl.ds(16, 16)]
    wcols = [w_lo[f] if f < 16 else w_hi[f - 16] for f in range(F)]
    bias = bv[...]
    iota32 = lax.iota(jnp.int32, 16) * F

    def group(g, carry):
        idx0 = g * (16 * F) + iota32
        acc = bias
        for f in range(F):
            u = plsc.load_gather(ubuf, [idx0 + f])
            iv = plsc.load_gather(ibuf, [idx0 + f])
            acc = acc + u * iv * wcols[f]
        outv[pl.ds(g * 16, 16)] = acc
        return carry

    lax.fori_loop(0, bpw // 16, group, 0)
    pltpu.sync_copy(outv, out_hbm.at[pl.ds(wid * bpw, bpw)])


def _make_compute_kernel():
    mesh = plsc.VectorSubcoreMesh(core_axis_name="c", subcore_axis_name="s",
                                  num_cores=2, num_subcores=16)
    bpw = B // NW
    return pl.kernel(
        _compute_body,
        out_type=jax.ShapeDtypeStruct((B,), jnp.float32),
        mesh=mesh,
        compiler_params=pltpu.CompilerParams(needs_layout_passes=False),
        scratch_types=[
            pltpu.VMEM((bpw * F,), jnp.float32),
            pltpu.VMEM((bpw * F,), jnp.float32),
            pltpu.VMEM((F,), jnp.float32),
            pltpu.VMEM((16,), jnp.float32),
            pltpu.VMEM((bpw,), jnp.float32),
            pltpu.SemaphoreType.DMA,
        ],
    )


@functools.partial(jax.jit, static_argnames=())
def _gmf(user, item, euT, eiT, tail_u, tail_i, w_vec, bias_vec):
    gu, gi = _make_gather_kernel()(user, item, euT, eiT, tail_u, tail_i)
    return _make_compute_kernel()(gu, gi, w_vec, bias_vec)


def kernel(user, item, embed_user, embed_item, W, b):
    euT = embed_user.T
    eiT = embed_item.T
    tail_u = jnp.pad(euT[:, TAIL:], ((0, 0), (0, 64)))
    tail_i = jnp.pad(eiT[:, TAIL:], ((0, 0), (0, 64)))
    w_vec = W.reshape(F)
    bias_vec = jnp.broadcast_to(b, (16,))
    return _gmf(user, item, euT, eiT, tail_u, tail_i, w_vec, bias_vec)
